# baseline (speedup 1.0000x reference)
import functools

import jax
import jax.numpy as jnp
from jax import lax
from jax.experimental import pallas as pl
from jax.experimental.pallas import tpu as pltpu
from jax.experimental.pallas import tpu_sc as plsc

BATCH = 4096
SEQ = 200
D = 64
VOCAB = 1000000
NC, NS = 2, 16
NW = NC * NS
CPW = BATCH // NW
NBUF = 4
NROUNDS = SEQ // NBUF

_mesh = plsc.VectorSubcoreMesh(core_axis_name="c", subcore_axis_name="s")


@functools.partial(
    pl.kernel,
    mesh=_mesh,
    out_type=jax.ShapeDtypeStruct((SEQ, D, BATCH), jnp.float32),
    compiler_params=pltpu.CompilerParams(needs_layout_passes=False),
    scratch_types=[
        pltpu.VMEM((SEQ, CPW), jnp.int32),
        pltpu.VMEM((NBUF, CPW), jnp.int32),
        pltpu.VMEM((NBUF, CPW, 128), jnp.float32),
        pltpu.VMEM((NBUF, D, CPW), jnp.float32),
        pltpu.SemaphoreType.DMA((NBUF,)),
        pltpu.SemaphoreType.DMA((NBUF,)),
    ],
)
def _emb_lookup(ids_hbm, table_hbm, out_hbm, ids_v, idx_v, gbuf, obuf,
                gsem, ssem):
    wid = lax.axis_index("s") * NC + lax.axis_index("c")
    base = wid * CPW
    pltpu.sync_copy(ids_hbm.at[:, pl.ds(base, CPW)], ids_v)

    def prep(s, b):
        for g in range(CPW // 16):
            vec = ids_v.at[s][pl.ds(g * 16, 16)]
            idx_v.at[b][pl.ds(g * 16, 16)] = lax.shift_right_logical(vec, 1)

    def gather(b):
        pltpu.async_copy(table_hbm.at[idx_v.at[b]], gbuf.at[b], gsem.at[b])

    def gather_wait(b):
        pltpu.make_async_copy(table_hbm.at[idx_v.at[b]], gbuf.at[b],
                              gsem.at[b]).wait()

    def store(s, b):
        pltpu.async_copy(obuf.at[b], out_hbm.at[s, :, pl.ds(base, CPW)],
                         ssem.at[b])

    def store_wait(b):
        pltpu.make_async_copy(obuf.at[b], out_hbm.at[0, :, pl.ds(base, CPW)],
                              ssem.at[b]).wait()

    rowidx = [lax.iota(jnp.int32, 16) + (g * 16) for g in range(CPW // 16)]

    def transpose(s, b):
        gb = gbuf.at[b]
        ob = obuf.at[b]
        halves = [
            (ids_v.at[s][pl.ds(g * 16, 16)] & 1) * 64
            for g in range(CPW // 16)
        ]

        def dbody(d, carry):
            for g in range(CPW // 16):
                col = halves[g] + d
                val = plsc.load_gather(gb, [rowidx[g], col])
                ob.at[d][pl.ds(g * 16, 16)] = val
            return carry

        lax.fori_loop(0, D, dbody, 0)

    for b in range(NBUF):
        prep(b, b)
        gather(b)

    def body(r, carry):
        s0 = r * NBUF
        for b in range(NBUF):
            gather_wait(b)
            transpose(s0 + b, b)
            store(s0 + b, b)
        for b in range(NBUF):
            store_wait(b)
            prep(s0 + NBUF + b, b)
            gather(b)
        return carry

    lax.fori_loop(0, NROUNDS - 1, body, 0)

    s0 = (NROUNDS - 1) * NBUF
    for b in range(NBUF):
        gather_wait(b)
        transpose(s0 + b, b)
        store(s0 + b, b)
    for b in range(NBUF):
        store_wait(b)


def kernel(token_ids, embed_mat):
    ids_t = token_ids.T
    table2 = embed_mat.reshape(VOCAB // 2, 2 * D)
    out_t = _emb_lookup(ids_t, table2)
    return jnp.transpose(out_t, (2, 0, 1))
